# baseline (speedup 1.0000x reference)
import functools

import jax
import jax.numpy as jnp
from jax import lax
from jax.experimental import pallas as pl
from jax.experimental.pallas import tpu as pltpu
from jax.experimental.pallas import tpu_sc as plsc

BATCH = 4096
DIM = 128
LANES = 16
NCHUNK = DIM // LANES
BPW = BATCH // 32


def _ctx_kernel(
    sess_idx_hbm,
    subj_idx_hbm,
    sess_tab_hbm,
    subj_tab_hbm,
    sess_flag_hbm,
    subj_flag_hbm,
    out_hbm,
    sidx_v,
    bidx_v,
    soidx_v,
    boidx_v,
    sess_rows_v,
    subj_rows_v,
    sflag_v,
    bflag_v,
    sem_s,
    sem_b,
    sem_os,
    sem_ob,
):
    nc = 2
    wid = lax.axis_index("s") * nc + lax.axis_index("c")
    base = wid * BPW

    pltpu.sync_copy(sess_idx_hbm.at[pl.ds(base, BPW)], sidx_v)
    pltpu.sync_copy(subj_idx_hbm.at[pl.ds(base, BPW)], bidx_v)
    pltpu.sync_copy(sess_flag_hbm, sflag_v)
    pltpu.sync_copy(subj_flag_hbm, bflag_v)

    gs = pltpu.async_copy(sess_tab_hbm.at[sidx_v], sess_rows_v, sem_s)
    gb = pltpu.async_copy(subj_tab_hbm.at[bidx_v], subj_rows_v, sem_b)

    lane = lax.iota(jnp.int32, LANES)
    for j in range(NCHUNK):
        row = 2 * (base + j * LANES + lane)
        soidx_v[pl.ds(j * LANES, LANES)] = row
        boidx_v[pl.ds(j * LANES, LANES)] = row + 1

    sfl = [sflag_v[pl.ds(j * LANES, LANES)] for j in range(NCHUNK)]
    bfl = [bflag_v[pl.ds(j * LANES, LANES)] for j in range(NCHUNK)]

    gs.wait()

    def add_sess(i, _):
        for j in range(NCHUNK):
            sl = pl.ds(j * LANES, LANES)
            sess_rows_v[i, sl] = sess_rows_v[i, sl] + sfl[j]
        return _

    lax.fori_loop(0, BPW, add_sess, 0, unroll=2)
    os_dma = pltpu.async_copy(sess_rows_v, out_hbm.at[soidx_v], sem_os)

    gb.wait()

    def add_subj(i, _):
        for j in range(NCHUNK):
            sl = pl.ds(j * LANES, LANES)
            subj_rows_v[i, sl] = subj_rows_v[i, sl] + bfl[j]
        return _

    lax.fori_loop(0, BPW, add_subj, 0, unroll=2)
    ob_dma = pltpu.async_copy(subj_rows_v, out_hbm.at[boidx_v], sem_ob)

    os_dma.wait()
    ob_dma.wait()


@jax.jit
def kernel(session_idx, subject_idx, session_table, subject_table, session_flag, subject_flag):
    mesh = plsc.VectorSubcoreMesh(core_axis_name="c", subcore_axis_name="s")
    run = functools.partial(
        pl.kernel,
        mesh=mesh,
        out_type=jax.ShapeDtypeStruct((2 * BATCH, DIM), jnp.float32),
        scratch_types=[
            pltpu.VMEM((BPW,), jnp.int32),
            pltpu.VMEM((BPW,), jnp.int32),
            pltpu.VMEM((BPW,), jnp.int32),
            pltpu.VMEM((BPW,), jnp.int32),
            pltpu.VMEM((BPW, DIM), jnp.float32),
            pltpu.VMEM((BPW, DIM), jnp.float32),
            pltpu.VMEM((DIM,), jnp.float32),
            pltpu.VMEM((DIM,), jnp.float32),
            pltpu.SemaphoreType.DMA,
            pltpu.SemaphoreType.DMA,
            pltpu.SemaphoreType.DMA,
            pltpu.SemaphoreType.DMA,
        ],
    )(_ctx_kernel)
    flat = run(
        session_idx.astype(jnp.int32),
        subject_idx.astype(jnp.int32),
        session_table,
        subject_table,
        session_flag,
        subject_flag,
    )
    return flat.reshape(BATCH, 2, DIM)
